# baseline (speedup 1.0000x reference)
import functools

import jax
import jax.numpy as jnp
from jax import lax
from jax.experimental import pallas as pl
from jax.experimental.pallas import tpu as pltpu
from jax.experimental.pallas import tpu_sc as plsc

N_USERS = 5000
N_NODES = 10000
D = 256
HALF = 128
H1 = 128
N_EDGES = 320000

NC = 2
NS = 16
LANES = 16

CHUNK = 64
NCHUNKS = 316
EDGES_PER_TILE = NCHUNKS * CHUNK
NE_PAD = NS * EDGES_PER_TILE
EREC = 3 * CHUNK
ROWS_A = 632
ROWS_B = N_NODES - 15 * ROWS_A


NBUF = 4
GDEPTH = 3


def _sc_agg_body(xr_hbm, edata_hbm, out_hbm, agg_sh,
                 ed0, ed1, ed2, ed3, ix0, ix1, ix2, ix3,
                 dv0, dv1, dv2, dv3, rw0, rw1, rw2, rw3, sem_e, sem_g):
    c = lax.axis_index("c")
    s = lax.axis_index("s")
    eds = [ed0, ed1, ed2, ed3]
    ixs = [ix0, ix1, ix2, ix3]
    dvs = [dv0, dv1, dv2, dv3]
    rws = [rw0, rw1, rw2, rw3]

    def zero_row(r, _):
        for j in range(HALF // LANES):
            rw0[r, pl.ds(j * LANES, LANES)] = jnp.zeros((LANES,), jnp.float32)
        return 0
    lax.fori_loop(0, CHUNK, zero_row, 0)
    r0 = pl.multiple_of(s * ROWS_A, 8)

    nfa, rema = divmod(ROWS_A, CHUNK)
    nfb, remb = divmod(ROWS_B, CHUNK)

    @pl.when(s < 15)
    def _():
        for k in range(nfa):
            pltpu.sync_copy(rw0, agg_sh.at[pl.ds(r0 + k * CHUNK, CHUNK)])
        if rema:
            pltpu.sync_copy(rw0.at[pl.ds(0, rema)],
                            agg_sh.at[pl.ds(r0 + nfa * CHUNK, rema)])

    @pl.when(s == 15)
    def _():
        for k in range(nfb):
            pltpu.sync_copy(rw0, agg_sh.at[pl.ds(15 * ROWS_A + k * CHUNK, CHUNK)])
        if remb:
            pltpu.sync_copy(rw0.at[pl.ds(0, remb)],
                            agg_sh.at[pl.ds(15 * ROWS_A + nfb * CHUNK, remb)])
    plsc.subcore_barrier()

    base = s * NCHUNKS

    def ed_dma(k, slot):
        return pltpu.make_async_copy(
            edata_hbm.at[pl.ds((base + k) * EREC, EREC)],
            eds[slot].at[pl.ds(0, EREC)], sem_e.at[slot])

    def load_idx(slot):
        for j in range(CHUNK // LANES):
            ixs[slot][pl.ds(j * LANES, LANES)] = (
                eds[slot][pl.ds(j * LANES, LANES)].astype(jnp.int32) + c)
            dvs[slot][pl.ds(j * LANES, LANES)] = (
                eds[slot][pl.ds(CHUNK + j * LANES, LANES)].astype(jnp.int32))

    for i in range(GDEPTH):
        ed_dma(i, i).start()
        ed_dma(i, i).wait()
        load_idx(i)
        pltpu.async_copy(xr_hbm.at[ixs[i]], rws[i], sem_g.at[i])
    ed_dma(GDEPTH, GDEPTH).start()

    def super_body(kk, _):
        for i in range(NBUF):
            k = kk * NBUF + i
            pltpu.make_async_copy(xr_hbm.at[ixs[i]], rws[i], sem_g.at[i]).wait()

            def group_body(g, _, i=i):
                nvg = eds[i][pl.ds(2 * CHUNK + g * LANES, LANES)]
                for ri in range(LANES):
                    r = g * LANES + ri
                    nv = nvg[ri]
                    for j in range(HALF // LANES):
                        sl = pl.ds(j * LANES, LANES)
                        rws[i][r, sl] = rws[i][r, sl] * nv
                return 0
            lax.fori_loop(0, CHUNK // LANES, group_body, 0)

            kn = k + GDEPTH
            jn = (i + GDEPTH) % NBUF

            @pl.when(kn < NCHUNKS)
            def _(kn=kn, jn=jn):
                ed_dma(kn, jn).wait()
                load_idx(jn)
                pltpu.async_copy(xr_hbm.at[ixs[jn]], rws[jn], sem_g.at[jn])

            @pl.when(kn + 1 < NCHUNKS)
            def _(kn=kn, i=i):
                ed_dma(kn + 1, i).start()

            pltpu.sync_copy(rws[i], agg_sh.at[dvs[i]], add=True)
        return 0
    lax.fori_loop(0, NCHUNKS // NBUF, super_body, 0)

    plsc.subcore_barrier()

    @pl.when(s < 15)
    def _():
        pltpu.sync_copy(agg_sh.at[pl.ds(r0, ROWS_A)], out_hbm.at[c, pl.ds(r0, ROWS_A)])

    @pl.when(s == 15)
    def _():
        pltpu.sync_copy(agg_sh.at[pl.ds(15 * ROWS_A, ROWS_B)],
                        out_hbm.at[c, pl.ds(15 * ROWS_A, ROWS_B)])


_sc_agg = functools.partial(
    pl.kernel,
    mesh=plsc.VectorSubcoreMesh(core_axis_name="c", subcore_axis_name="s"),
    out_type=jax.ShapeDtypeStruct((NC, N_NODES, HALF), jnp.float32),
    scratch_types=(
        [pltpu.VMEM_SHARED((N_NODES, HALF), jnp.float32)]
        + [pltpu.VMEM((EREC + LANES,), jnp.float32)] * NBUF
        + [pltpu.VMEM((CHUNK,), jnp.int32)] * NBUF
        + [pltpu.VMEM((CHUNK,), jnp.int32)] * NBUF
        + [pltpu.VMEM((CHUNK, HALF), jnp.float32)] * NBUF
        + [pltpu.SemaphoreType.DMA((NBUF,)),
           pltpu.SemaphoreType.DMA((NBUF,))]
    ),
)(_sc_agg_body)


ROWB = 1000
NBLK_U = N_USERS // ROWB


def _enc_body(aL_ref, aR_ref, Wr_ref, Wu_ref, Wi_ref, Q_ref, out_ref):
    b = pl.program_id(0)
    aL = aL_ref[0]
    aR = aR_ref[0]
    Wr = Wr_ref[...]
    feats = jnp.maximum(
        jnp.dot(aL, Wr[:HALF], preferred_element_type=jnp.float32)
        + jnp.dot(aR, Wr[HALF:], preferred_element_type=jnp.float32), 0.0)
    is_user = b < NBLK_U
    W2 = jnp.where(is_user, Wu_ref[...], Wi_ref[...])
    h = jnp.maximum(jnp.dot(feats, W2, preferred_element_type=jnp.float32), 0.0)
    hq = jnp.dot(h, Q_ref[...], preferred_element_type=jnp.float32)
    out_ref[...] = jnp.where(is_user, hq, h)


def _encode(agg2, W_rgc, W_u, W_i, Q):
    wspec = lambda shape: pl.BlockSpec(shape, lambda b: (0, 0))
    return pl.pallas_call(
        _enc_body,
        grid=(N_NODES // ROWB,),
        in_specs=[
            pl.BlockSpec((1, ROWB, HALF), lambda b: (0, b, 0)),
            pl.BlockSpec((1, ROWB, HALF), lambda b: (1, b, 0)),
            wspec((D, D)),
            wspec((D, H1)),
            wspec((D, H1)),
            wspec((H1, H1)),
        ],
        out_specs=pl.BlockSpec((ROWB, H1), lambda b: (b, 0)),
        out_shape=jax.ShapeDtypeStruct((N_NODES, H1), jnp.float32),
    )(agg2, agg2, W_rgc, W_u, W_i, Q)


NPAD = 5120


def _dec_body(u_ref, v_ref, out_ref):
    z = lax.dot_general(u_ref[...], v_ref[...], (((1,), (1,)), ((), ())),
                        preferred_element_type=jnp.float32)
    sig = 1.0 / (1.0 + jnp.exp(-z))
    zfull = jnp.concatenate(
        [sig, jnp.zeros((ROWB, NPAD - N_USERS), jnp.float32)], axis=1)
    zsh = pltpu.roll(zfull, 8, 1)
    par = (lax.broadcasted_iota(jnp.int32, (ROWB, 1), 0) % 2) == 1
    out_ref[...] = jnp.where(par, zsh, zfull)


def _decode(uqi):
    return pl.pallas_call(
        _dec_body,
        grid=(NBLK_U,),
        in_specs=[
            pl.BlockSpec((ROWB, H1), lambda i: (i, 0)),
            pl.BlockSpec((N_USERS, H1), lambda i: (1, 0)),
        ],
        out_specs=pl.BlockSpec((ROWB, NPAD), lambda i: (i, 0)),
        out_shape=jax.ShapeDtypeStruct((N_USERS, NPAD), jnp.float32),
    )(uqi, uqi)


GRP = N_USERS // 8
NSUP = N_USERS // 16
SUPW = 16 * N_USERS
TGRP = 10


def _sc_lin_body(z3_hbm, out_hbm, buf, lin, sem_r, sem_w):
    c = lax.axis_index("c")
    s = lax.axis_index("s")
    w = s * NC + c

    def vcopy_group(lin_base):
        for pr in range(4):
            te, to = 2 * pr, 2 * pr + 1
            eoff = lin_base + 5000 * te
            ooff = lin_base + 5000 * to - 8

            def ev(j, _, te=te, eoff=eoff):
                sl = pl.multiple_of(j * LANES, LANES)
                lin[pl.ds(eoff + sl, LANES)] = buf[te, pl.ds(sl, LANES)]
                return 0
            lax.fori_loop(0, 312, ev, 0, unroll=8)
            mask = lax.iota(jnp.int32, LANES) < 8
            cmb = jnp.where(mask, buf[te, pl.ds(4992, LANES)],
                            buf[to, pl.ds(0, LANES)])
            lin[pl.ds(ooff, LANES)] = cmb

            def od(j, _, to=to, ooff=ooff):
                sl = pl.multiple_of(j * LANES, LANES)
                lin[pl.ds(ooff + sl, LANES)] = buf[to, pl.ds(sl, LANES)]
                return 0
            lax.fori_loop(1, 313, od, 0, unroll=8)

    def wr_dma(p, nwords):
        return pltpu.make_async_copy(
            lin.at[pl.ds(0, nwords)],
            out_hbm.at[0, 0, pl.ds(p * SUPW, nwords)], sem_w)

    TAIL1 = (SUPW // 2) // 128 * 128
    TAIL2 = SUPW // 2 - TAIL1

    def tail_dmas():
        return (pltpu.make_async_copy(lin.at[pl.ds(0, TAIL1)],
                                      out_hbm.at[0, 0, pl.ds(NSUP * SUPW, TAIL1)],
                                      sem_w),
                pltpu.make_async_copy(
                    lin.at[pl.ds(TAIL1, TAIL2)],
                    out_hbm.at[0, 0, pl.ds(NSUP * SUPW + TAIL1, TAIL2)],
                    sem_w))

    for t in range(TGRP):
        p = w + 32 * t

        @pl.when(p <= NSUP)
        def _(p=p, t=t):
            pltpu.sync_copy(z3_hbm.at[2 * p], buf)
            if t > 0:
                pltpu.make_async_copy(
                    lin, out_hbm.at[0, 0, pl.ds((p - 32) * SUPW, SUPW)],
                    sem_w).wait()
            vcopy_group(0)

            @pl.when(p < NSUP)
            def _(p=p):
                pltpu.sync_copy(z3_hbm.at[2 * p + 1], buf)
                vcopy_group(40000)
                wr_dma(p, SUPW).start()

            @pl.when(p == NSUP)
            def _(p=p):
                for dma in tail_dmas():
                    dma.start()

    for t in range(TGRP):
        p = w + 32 * t

        @pl.when(jnp.logical_and(p < NSUP, p + 32 > NSUP))
        def _(p=p):
            wr_dma(p, SUPW).wait()

        @pl.when(p == NSUP)
        def _(p=p):
            for dma in tail_dmas():
                dma.wait()


_sc_lin = functools.partial(
    pl.kernel,
    mesh=plsc.VectorSubcoreMesh(core_axis_name="c", subcore_axis_name="s"),
    out_type=jax.ShapeDtypeStruct((1, 1, N_USERS * N_USERS), jnp.float32),
    scratch_types=[
        pltpu.VMEM((8, NPAD), jnp.float32),
        pltpu.VMEM((SUPW,), jnp.float32),
        pltpu.SemaphoreType.DMA,
        pltpu.SemaphoreType.DMA,
    ],
)(_sc_lin_body)


def kernel(x, edge_index, edge_norm, W_rgc, W_u, W_i, Q):
    xr = x.reshape(N_NODES * 2, HALF)
    pad = NE_PAD - N_EDGES
    src2 = jnp.concatenate([edge_index[0] * 2, jnp.zeros((pad,), jnp.int32)])
    dstp = jnp.concatenate([edge_index[1],
                            jnp.arange(pad, dtype=jnp.int32) % N_NODES])
    normp = jnp.concatenate([edge_norm, jnp.zeros((pad,), jnp.float32)])
    edata = jnp.stack([
        src2.astype(jnp.float32).reshape(-1, CHUNK),
        dstp.astype(jnp.float32).reshape(-1, CHUNK),
        normp.reshape(-1, CHUNK),
    ], axis=1).reshape(-1)
    agg2 = _sc_agg(xr, edata)
    uqi = _encode(agg2, W_rgc, W_u, W_i, Q)
    zp = _decode(uqi)
    z3 = zp.reshape(GRP, 8, NPAD)
    out = _sc_lin(z3)
    return out.reshape(N_USERS * N_USERS, 1)


# speedup vs baseline: 1.0459x; 1.0459x over previous
import functools

import jax
import jax.numpy as jnp
from jax import lax
from jax.experimental import pallas as pl
from jax.experimental.pallas import tpu as pltpu
from jax.experimental.pallas import tpu_sc as plsc

N_USERS = 5000
N_NODES = 10000
D = 256
HALF = 128
H1 = 128
N_EDGES = 320000

NC = 2
NS = 16
LANES = 16

CHUNK = 64
NCHUNKS = 315
EDGES_PER_TILE = NCHUNKS * CHUNK
NE_PAD = NS * EDGES_PER_TILE
EREC = 3 * CHUNK
ROWS_A = 632
ROWS_B = N_NODES - 15 * ROWS_A


NBUF = 5
GDEPTH = 4


def _sc_agg_body(xr_hbm, edata_hbm, out_hbm, agg_sh,
                 ed0, ed1, ed2, ed3, ed4, ix0, ix1, ix2, ix3, ix4,
                 dv0, dv1, dv2, dv3, dv4, rw0, rw1, rw2, rw3, rw4, sem_e, sem_g):
    c = lax.axis_index("c")
    s = lax.axis_index("s")
    eds = [ed0, ed1, ed2, ed3, ed4]
    ixs = [ix0, ix1, ix2, ix3, ix4]
    dvs = [dv0, dv1, dv2, dv3, dv4]
    rws = [rw0, rw1, rw2, rw3, rw4]

    def zero_row(r, _):
        for j in range(HALF // LANES):
            rw0[r, pl.ds(j * LANES, LANES)] = jnp.zeros((LANES,), jnp.float32)
        return 0
    lax.fori_loop(0, CHUNK, zero_row, 0)
    r0 = pl.multiple_of(s * ROWS_A, 8)

    nfa, rema = divmod(ROWS_A, CHUNK)
    nfb, remb = divmod(ROWS_B, CHUNK)

    @pl.when(s < 15)
    def _():
        for k in range(nfa):
            pltpu.sync_copy(rw0, agg_sh.at[pl.ds(r0 + k * CHUNK, CHUNK)])
        if rema:
            pltpu.sync_copy(rw0.at[pl.ds(0, rema)],
                            agg_sh.at[pl.ds(r0 + nfa * CHUNK, rema)])

    @pl.when(s == 15)
    def _():
        for k in range(nfb):
            pltpu.sync_copy(rw0, agg_sh.at[pl.ds(15 * ROWS_A + k * CHUNK, CHUNK)])
        if remb:
            pltpu.sync_copy(rw0.at[pl.ds(0, remb)],
                            agg_sh.at[pl.ds(15 * ROWS_A + nfb * CHUNK, remb)])
    plsc.subcore_barrier()

    base = s * NCHUNKS

    def ed_dma(k, slot):
        return pltpu.make_async_copy(
            edata_hbm.at[pl.ds((base + k) * EREC, EREC)],
            eds[slot].at[pl.ds(0, EREC)], sem_e.at[slot])

    def load_idx(slot):
        for j in range(CHUNK // LANES):
            ixs[slot][pl.ds(j * LANES, LANES)] = (
                eds[slot][pl.ds(j * LANES, LANES)].astype(jnp.int32) + c)
            dvs[slot][pl.ds(j * LANES, LANES)] = (
                eds[slot][pl.ds(CHUNK + j * LANES, LANES)].astype(jnp.int32))

    for i in range(GDEPTH):
        ed_dma(i, i).start()
        ed_dma(i, i).wait()
        load_idx(i)
        pltpu.async_copy(xr_hbm.at[ixs[i]], rws[i], sem_g.at[i])
    ed_dma(GDEPTH, GDEPTH).start()

    def super_body(kk, _):
        for i in range(NBUF):
            k = kk * NBUF + i
            pltpu.make_async_copy(xr_hbm.at[ixs[i]], rws[i], sem_g.at[i]).wait()

            def group_body(g, _, i=i):
                nvg = eds[i][pl.ds(2 * CHUNK + g * LANES, LANES)]
                for ri in range(LANES):
                    r = g * LANES + ri
                    nv = nvg[ri]
                    for j in range(HALF // LANES):
                        sl = pl.ds(j * LANES, LANES)
                        rws[i][r, sl] = rws[i][r, sl] * nv
                return 0
            lax.fori_loop(0, CHUNK // LANES, group_body, 0)

            kn = k + GDEPTH
            jn = (i + GDEPTH) % NBUF

            @pl.when(kn < NCHUNKS)
            def _(kn=kn, jn=jn):
                ed_dma(kn, jn).wait()
                load_idx(jn)
                pltpu.async_copy(xr_hbm.at[ixs[jn]], rws[jn], sem_g.at[jn])

            @pl.when(kn + 1 < NCHUNKS)
            def _(kn=kn, i=i):
                ed_dma(kn + 1, i).start()

            pltpu.sync_copy(rws[i], agg_sh.at[dvs[i]], add=True)
        return 0
    lax.fori_loop(0, NCHUNKS // NBUF, super_body, 0)

    plsc.subcore_barrier()

    @pl.when(s < 15)
    def _():
        pltpu.sync_copy(agg_sh.at[pl.ds(r0, ROWS_A)], out_hbm.at[c, pl.ds(r0, ROWS_A)])

    @pl.when(s == 15)
    def _():
        pltpu.sync_copy(agg_sh.at[pl.ds(15 * ROWS_A, ROWS_B)],
                        out_hbm.at[c, pl.ds(15 * ROWS_A, ROWS_B)])


_sc_agg = functools.partial(
    pl.kernel,
    mesh=plsc.VectorSubcoreMesh(core_axis_name="c", subcore_axis_name="s"),
    out_type=jax.ShapeDtypeStruct((NC, N_NODES, HALF), jnp.float32),
    scratch_types=(
        [pltpu.VMEM_SHARED((N_NODES, HALF), jnp.float32)]
        + [pltpu.VMEM((EREC + LANES,), jnp.float32)] * NBUF
        + [pltpu.VMEM((CHUNK,), jnp.int32)] * NBUF
        + [pltpu.VMEM((CHUNK,), jnp.int32)] * NBUF
        + [pltpu.VMEM((CHUNK, HALF), jnp.float32)] * NBUF
        + [pltpu.SemaphoreType.DMA((NBUF,)),
           pltpu.SemaphoreType.DMA((NBUF,))]
    ),
)(_sc_agg_body)


ROWB = 1000
NBLK_U = N_USERS // ROWB


def _enc_body(aL_ref, aR_ref, Wr_ref, Wu_ref, Wi_ref, Q_ref, out_ref):
    b = pl.program_id(0)
    aL = aL_ref[0]
    aR = aR_ref[0]
    Wr = Wr_ref[...]
    feats = jnp.maximum(
        jnp.dot(aL, Wr[:HALF], preferred_element_type=jnp.float32)
        + jnp.dot(aR, Wr[HALF:], preferred_element_type=jnp.float32), 0.0)
    is_user = b < NBLK_U
    W2 = jnp.where(is_user, Wu_ref[...], Wi_ref[...])
    h = jnp.maximum(jnp.dot(feats, W2, preferred_element_type=jnp.float32), 0.0)
    hq = jnp.dot(h, Q_ref[...], preferred_element_type=jnp.float32)
    out_ref[...] = jnp.where(is_user, hq, h)


def _encode(agg2, W_rgc, W_u, W_i, Q):
    wspec = lambda shape: pl.BlockSpec(shape, lambda b: (0, 0))
    return pl.pallas_call(
        _enc_body,
        grid=(N_NODES // ROWB,),
        in_specs=[
            pl.BlockSpec((1, ROWB, HALF), lambda b: (0, b, 0)),
            pl.BlockSpec((1, ROWB, HALF), lambda b: (1, b, 0)),
            wspec((D, D)),
            wspec((D, H1)),
            wspec((D, H1)),
            wspec((H1, H1)),
        ],
        out_specs=pl.BlockSpec((ROWB, H1), lambda b: (b, 0)),
        out_shape=jax.ShapeDtypeStruct((N_NODES, H1), jnp.float32),
    )(agg2, agg2, W_rgc, W_u, W_i, Q)


NPAD = 5120


def _dec_body(u_ref, v_ref, out_ref):
    z = lax.dot_general(u_ref[...], v_ref[...], (((1,), (1,)), ((), ())),
                        preferred_element_type=jnp.float32)
    sig = 1.0 / (1.0 + jnp.exp(-z))
    zfull = jnp.concatenate(
        [sig, jnp.zeros((ROWB, NPAD - N_USERS), jnp.float32)], axis=1)
    zsh = pltpu.roll(zfull, 8, 1)
    par = (lax.broadcasted_iota(jnp.int32, (ROWB, 1), 0) % 2) == 1
    out_ref[...] = jnp.where(par, zsh, zfull)


def _decode(uqi):
    return pl.pallas_call(
        _dec_body,
        grid=(NBLK_U,),
        in_specs=[
            pl.BlockSpec((ROWB, H1), lambda i: (i, 0)),
            pl.BlockSpec((N_USERS, H1), lambda i: (1, 0)),
        ],
        out_specs=pl.BlockSpec((ROWB, NPAD), lambda i: (i, 0)),
        out_shape=jax.ShapeDtypeStruct((N_USERS, NPAD), jnp.float32),
    )(uqi, uqi)


GRP = N_USERS // 8
NSUP = N_USERS // 16
SUPW = 16 * N_USERS
TGRP = 10


def _sc_lin_body(z3_hbm, out_hbm, buf, lin, sem_r, sem_w):
    c = lax.axis_index("c")
    s = lax.axis_index("s")
    w = s * NC + c

    def vcopy_group(lin_base):
        for pr in range(4):
            te, to = 2 * pr, 2 * pr + 1
            eoff = lin_base + 5000 * te
            ooff = lin_base + 5000 * to - 8

            def ev(j, _, te=te, eoff=eoff):
                sl = pl.multiple_of(j * LANES, LANES)
                lin[pl.ds(eoff + sl, LANES)] = buf[te, pl.ds(sl, LANES)]
                return 0
            lax.fori_loop(0, 312, ev, 0, unroll=8)
            mask = lax.iota(jnp.int32, LANES) < 8
            cmb = jnp.where(mask, buf[te, pl.ds(4992, LANES)],
                            buf[to, pl.ds(0, LANES)])
            lin[pl.ds(ooff, LANES)] = cmb

            def od(j, _, to=to, ooff=ooff):
                sl = pl.multiple_of(j * LANES, LANES)
                lin[pl.ds(ooff + sl, LANES)] = buf[to, pl.ds(sl, LANES)]
                return 0
            lax.fori_loop(1, 313, od, 0, unroll=8)

    def wr_dma(p, nwords):
        return pltpu.make_async_copy(
            lin.at[pl.ds(0, nwords)],
            out_hbm.at[0, 0, pl.ds(p * SUPW, nwords)], sem_w)

    TAIL1 = (SUPW // 2) // 128 * 128
    TAIL2 = SUPW // 2 - TAIL1

    def tail_dmas():
        return (pltpu.make_async_copy(lin.at[pl.ds(0, TAIL1)],
                                      out_hbm.at[0, 0, pl.ds(NSUP * SUPW, TAIL1)],
                                      sem_w),
                pltpu.make_async_copy(
                    lin.at[pl.ds(TAIL1, TAIL2)],
                    out_hbm.at[0, 0, pl.ds(NSUP * SUPW + TAIL1, TAIL2)],
                    sem_w))

    for t in range(TGRP):
        p = w + 32 * t

        @pl.when(p <= NSUP)
        def _(p=p, t=t):
            pltpu.sync_copy(z3_hbm.at[2 * p], buf)
            if t > 0:
                pltpu.make_async_copy(
                    lin, out_hbm.at[0, 0, pl.ds((p - 32) * SUPW, SUPW)],
                    sem_w).wait()
            vcopy_group(0)

            @pl.when(p < NSUP)
            def _(p=p):
                pltpu.sync_copy(z3_hbm.at[2 * p + 1], buf)
                vcopy_group(40000)
                wr_dma(p, SUPW).start()

            @pl.when(p == NSUP)
            def _(p=p):
                for dma in tail_dmas():
                    dma.start()

    for t in range(TGRP):
        p = w + 32 * t

        @pl.when(jnp.logical_and(p < NSUP, p + 32 > NSUP))
        def _(p=p):
            wr_dma(p, SUPW).wait()

        @pl.when(p == NSUP)
        def _(p=p):
            for dma in tail_dmas():
                dma.wait()


_sc_lin = functools.partial(
    pl.kernel,
    mesh=plsc.VectorSubcoreMesh(core_axis_name="c", subcore_axis_name="s"),
    out_type=jax.ShapeDtypeStruct((1, 1, N_USERS * N_USERS), jnp.float32),
    scratch_types=[
        pltpu.VMEM((8, NPAD), jnp.float32),
        pltpu.VMEM((SUPW,), jnp.float32),
        pltpu.SemaphoreType.DMA,
        pltpu.SemaphoreType.DMA,
    ],
)(_sc_lin_body)


def kernel(x, edge_index, edge_norm, W_rgc, W_u, W_i, Q):
    xr = x.reshape(N_NODES * 2, HALF)
    pad = NE_PAD - N_EDGES
    src2 = jnp.concatenate([edge_index[0] * 2, jnp.zeros((pad,), jnp.int32)])
    dstp = jnp.concatenate([edge_index[1],
                            jnp.arange(pad, dtype=jnp.int32) % N_NODES])
    normp = jnp.concatenate([edge_norm, jnp.zeros((pad,), jnp.float32)])
    edata = jnp.stack([
        src2.astype(jnp.float32).reshape(-1, CHUNK),
        dstp.astype(jnp.float32).reshape(-1, CHUNK),
        normp.reshape(-1, CHUNK),
    ], axis=1).reshape(-1)
    agg2 = _sc_agg(xr, edata)
    uqi = _encode(agg2, W_rgc, W_u, W_i, Q)
    zp = _decode(uqi)
    z3 = zp.reshape(GRP, 8, NPAD)
    out = _sc_lin(z3)
    return out.reshape(N_USERS * N_USERS, 1)
